# baseline (speedup 1.0000x reference)
import jax
import jax.numpy as jnp
from jax import lax
from jax.experimental import pallas as pl
from jax.experimental.pallas import tpu as pltpu
from jax.experimental.pallas import tpu_sc as plsc

N = 50000
E = 800000
HID = 64

NC, NS = 2, 16
PAY = 32
EXW = 8
SC_CHUNK = 512
N_CH_SCAT = 98
E_PAD = NS * N_CH_SCAT * SC_CHUNK
G_CHUNK = 512
N_CH_GATH = E_PAD // (NC * NS * G_CHUNK)
N_ACC = N + NS
ROWS_PT = N_ACC // NS

BLK_E = 8192
BLK_N = 2000

_SC_PARAMS = pltpu.CompilerParams(
    use_tc_tiling_on_sc=False, needs_layout_passes=False)


def _sc_gather_body(src_hbm, dst_hbm, xpad_hbm, zeros_hbm,
                    s_hbm, ex_hbm, acc, ibs, ibd, rbuf, sbuf, sem):
    cid = lax.axis_index("c")
    sid = lax.axis_index("s")
    wid = sid * NC + cid

    pltpu.sync_copy(zeros_hbm, acc.at[pl.ds(sid * ROWS_PT, ROWS_PT)])
    plsc.subcore_barrier()
    lanes = lax.iota(jnp.int32, 16)
    zl = jnp.zeros((16,), jnp.int32)

    def body(g, carry):
        ch = wid * N_CH_GATH + g
        r0 = ch * (G_CHUNK // 128)
        pltpu.sync_copy(src_hbm.at[pl.ds(r0, G_CHUNK // 128)], ibs)
        pltpu.sync_copy(dst_hbm.at[pl.ds(r0, G_CHUNK // 128)], ibd)
        for j in range(G_CHUNK // 128):
            pltpu.async_copy(xpad_hbm.at[ibs.at[j]],
                             rbuf.at[pl.ds(j * 128, 128)], sem).wait()
        for t in range(G_CHUNK // 16):
            sbuf[pl.ds(t * 16, 16)] = plsc.load_gather(
                rbuf, [t * 16 + lanes, zl])
        pltpu.sync_copy(sbuf, s_hbm.at[pl.ds(ch * G_CHUNK, G_CHUNK)])
        for j in range(G_CHUNK // 128):
            pltpu.sync_copy(rbuf.at[pl.ds(j * 128, 128)],
                            acc.at[ibd.at[j]], add=True)
        return carry

    lax.fori_loop(0, N_CH_GATH, body, 0)
    plsc.subcore_barrier()
    pltpu.sync_copy(acc.at[pl.ds(sid * ROWS_PT, ROWS_PT)],
                    ex_hbm.at[cid].at[pl.ds(sid * ROWS_PT, ROWS_PT)])


def _sc_scatter_body(p_hbm, dst_hbm, zeros_hbm, out_hbm, acc, pbuf, ibuf):
    cid = lax.axis_index("c")
    sid = lax.axis_index("s")

    pltpu.sync_copy(zeros_hbm, acc.at[pl.ds(sid * ROWS_PT, ROWS_PT)])
    plsc.subcore_barrier()

    def body(g, carry):
        e0 = (sid * N_CH_SCAT + g) * SC_CHUNK
        pltpu.sync_copy(p_hbm.at[cid].at[pl.ds(e0, SC_CHUNK)], pbuf)
        pltpu.sync_copy(dst_hbm.at[pl.ds(e0 // 128, SC_CHUNK // 128)], ibuf)
        for j in range(SC_CHUNK // 128):
            pltpu.sync_copy(pbuf.at[pl.ds(j * 128, 128)],
                            acc.at[ibuf.at[j]], add=True)
        return carry

    lax.fori_loop(0, N_CH_SCAT, body, 0)
    plsc.subcore_barrier()
    pltpu.sync_copy(acc.at[pl.ds(sid * ROWS_PT, ROWS_PT)],
                    out_hbm.at[cid].at[pl.ds(sid * ROWS_PT, ROWS_PT)])


_sc_cache = {}


def _sc_kernels():
    if "k" not in _sc_cache:
        mesh = plsc.VectorSubcoreMesh(
            core_axis_name="c", subcore_axis_name="s",
            num_cores=NC, num_subcores=NS)
        gather = pl.kernel(
            _sc_gather_body,
            out_type=[
                jax.ShapeDtypeStruct((E_PAD,), jnp.float32),
                jax.ShapeDtypeStruct((NC, N_ACC, EXW), jnp.float32),
            ],
            mesh=mesh,
            compiler_params=_SC_PARAMS,
            scratch_types=[
                pltpu.VMEM_SHARED((N_ACC, EXW), jnp.float32),
                pltpu.VMEM((G_CHUNK // 128, 128), jnp.int32),
                pltpu.VMEM((G_CHUNK // 128, 128), jnp.int32),
                pltpu.VMEM((G_CHUNK, EXW), jnp.float32),
                pltpu.VMEM((G_CHUNK,), jnp.float32),
                pltpu.SemaphoreType.DMA,
            ])
        scatter = pl.kernel(
            _sc_scatter_body,
            out_type=jax.ShapeDtypeStruct((NC, N_ACC, PAY), jnp.float32),
            mesh=mesh,
            compiler_params=_SC_PARAMS,
            scratch_types=[
                pltpu.VMEM_SHARED((N_ACC, PAY), jnp.float32),
                pltpu.VMEM((SC_CHUNK, PAY), jnp.float32),
                pltpu.VMEM((SC_CHUNK // 128, 128), jnp.int32),
            ])
        _sc_cache["k"] = (gather, scatter)
    return _sc_cache["k"]


Q = E_PAD // 4
BLK_R = 2048


def _edge_body(ea0, ea1, ea2, ea3, sp_ref, w1_ref, b1_ref, p_ref):
    lo, hi = [], []
    for ea_ref in (ea0, ea1, ea2, ea3):
        z = jnp.dot(ea_ref[...], w1_ref[...],
                    preferred_element_type=jnp.float32)
        z = jnp.maximum(z + b1_ref[...], 0.0)
        lo.append(z[:, :PAY])
        hi.append(z[:, PAY:])
    sp = sp_ref[...]
    p_ref[0] = jnp.concatenate(lo, axis=1) * sp
    p_ref[1] = jnp.concatenate(hi, axis=1) * sp


def _tc_edge(ea, spack, W1, b1):
    nb = Q // BLK_R
    ea_specs = [pl.BlockSpec((BLK_R, 16), lambda i, k=k: (k * nb + i, 0))
                for k in range(4)]
    return pl.pallas_call(
        _edge_body,
        grid=(nb,),
        in_specs=ea_specs + [
            pl.BlockSpec((BLK_R, 128), lambda i: (i, 0)),
            pl.BlockSpec((16, HID), lambda i: (0, 0)),
            pl.BlockSpec((1, HID), lambda i: (0, 0)),
        ],
        out_specs=pl.BlockSpec((NC, BLK_R, 128), lambda i: (0, i, 0)),
        out_shape=jax.ShapeDtypeStruct((NC, Q, 128), jnp.float32),
    )(ea, ea, ea, ea, spack, W1, b1)


def _d1_body(a0_ref, a1_ref, ex0_ref, ex1_ref, x_ref, w2_ref, b2_ref,
             wr_ref, bc_ref, pre_ref, sum_ref, sq_ref):
    A = jnp.concatenate([a0_ref[...], a1_ref[...]], axis=1)
    ex = ex0_ref[...] + ex1_ref[...]
    cs = ex[:, 0:1]
    deg = ex[:, 1:2]
    agg = jnp.dot(A, w2_ref[...], preferred_element_type=jnp.float32)
    agg = (agg + cs * b2_ref[...]) / jnp.maximum(deg, 1.0)
    pre = agg + x_ref[...] * wr_ref[...] + bc_ref[...]
    pre_ref[...] = pre

    @pl.when(pl.program_id(0) == 0)
    def _():
        sum_ref[...] = jnp.zeros_like(sum_ref)
        sq_ref[...] = jnp.zeros_like(sq_ref)

    sum_ref[...] += jnp.sum(pre, axis=0, keepdims=True)
    sq_ref[...] += jnp.sum(pre * pre, axis=0, keepdims=True)


def _tc_d1(A0, A1, EX0, EX1, x, W2, b2, Wr, bc):
    return pl.pallas_call(
        _d1_body,
        grid=(N // BLK_N,),
        in_specs=[
            pl.BlockSpec((BLK_N, PAY), lambda i: (i, 0)),
            pl.BlockSpec((BLK_N, PAY), lambda i: (i, 0)),
            pl.BlockSpec((BLK_N, EXW), lambda i: (i, 0)),
            pl.BlockSpec((BLK_N, EXW), lambda i: (i, 0)),
            pl.BlockSpec((BLK_N, 1), lambda i: (i, 0)),
            pl.BlockSpec((HID, HID), lambda i: (0, 0)),
            pl.BlockSpec((1, HID), lambda i: (0, 0)),
            pl.BlockSpec((1, HID), lambda i: (0, 0)),
            pl.BlockSpec((1, HID), lambda i: (0, 0)),
        ],
        out_specs=[
            pl.BlockSpec((BLK_N, HID), lambda i: (i, 0)),
            pl.BlockSpec((1, HID), lambda i: (0, 0)),
            pl.BlockSpec((1, HID), lambda i: (0, 0)),
        ],
        out_shape=[
            jax.ShapeDtypeStruct((N, HID), jnp.float32),
            jax.ShapeDtypeStruct((1, HID), jnp.float32),
            jax.ShapeDtypeStruct((1, HID), jnp.float32),
        ],
    )(A0, A1, EX0, EX1, x, W2, b2, Wr, bc)


def _d2_body(pre_ref, sum_ref, sq_ref, g_ref, be_ref, wl1_ref, bl1_ref,
             wl2_ref, bl2_ref, y_ref):
    mean = sum_ref[...] * (1.0 / N)
    var = sq_ref[...] * (1.0 / N) - mean * mean
    inv = lax.rsqrt(var + 1e-5)
    o = (pre_ref[...] - mean) * (inv * g_ref[...]) + be_ref[...]
    o = jnp.maximum(o, 0.0)
    l1 = jnp.dot(o, wl1_ref[...], preferred_element_type=jnp.float32)
    l1 = jax.nn.sigmoid(l1 + bl1_ref[...])
    y_ref[...] = (jnp.dot(l1, wl2_ref[...], preferred_element_type=jnp.float32)
                  + bl2_ref[...])


def _tc_d2(pre, s1, s2, gamma, beta, Wl1, bl1, Wl2, bl2):
    return pl.pallas_call(
        _d2_body,
        grid=(N // BLK_N,),
        in_specs=[
            pl.BlockSpec((BLK_N, HID), lambda i: (i, 0)),
            pl.BlockSpec((1, HID), lambda i: (0, 0)),
            pl.BlockSpec((1, HID), lambda i: (0, 0)),
            pl.BlockSpec((1, HID), lambda i: (0, 0)),
            pl.BlockSpec((1, HID), lambda i: (0, 0)),
            pl.BlockSpec((HID, HID // 2), lambda i: (0, 0)),
            pl.BlockSpec((1, HID // 2), lambda i: (0, 0)),
            pl.BlockSpec((HID // 2, 1), lambda i: (0, 0)),
            pl.BlockSpec((1, 1), lambda i: (0, 0)),
        ],
        out_specs=pl.BlockSpec((BLK_N, 1), lambda i: (i, 0)),
        out_shape=jax.ShapeDtypeStruct((N, 1), jnp.float32),
    )(pre, s1, s2, gamma, beta, Wl1, bl1, Wl2, bl2)


def kernel(x, edge_index, edge_attr, W1, b1, W2, b2, Wr, b_conv, gamma, beta,
           Wl1, bl1, Wl2, bl2):
    pad = E_PAD - E
    src = edge_index[0].astype(jnp.int32)
    dst = edge_index[1].astype(jnp.int32)
    src_p = jnp.concatenate([src, jnp.zeros((pad,), jnp.int32)])
    src_p = src_p.reshape(E_PAD // 128, 128)
    dst_p = jnp.concatenate([dst, jnp.full((pad,), N, jnp.int32)])
    dst_p = dst_p.reshape(E_PAD // 128, 128)
    xpad = jnp.concatenate(
        [x, jnp.ones((N, 1), jnp.float32), jnp.zeros((N, EXW - 2), jnp.float32)],
        axis=1)
    ea_p = jnp.concatenate(
        [edge_attr, jnp.zeros((pad, 16), jnp.float32)], axis=0)
    zeros8 = jnp.zeros((ROWS_PT, EXW), jnp.float32)
    zeros32 = jnp.zeros((ROWS_PT, PAY), jnp.float32)

    dst_sig = jnp.concatenate([dst, jnp.full((pad,), N, jnp.int32)])
    dst_sig = dst_sig.reshape(4, Q).T.reshape(E_PAD // 128, 128)

    sc_gather, sc_scatter = _sc_kernels()
    s_flat, ex = sc_gather(src_p, dst_p, xpad, zeros8)
    spack = jnp.broadcast_to(
        s_flat.reshape(4, Q).T[:, :, None], (Q, 4, PAY)).reshape(Q, 128)
    P = _tc_edge(ea_p, spack, W1, b1.reshape(1, HID))
    acc = sc_scatter(P.reshape(NC, E_PAD, PAY), dst_sig, zeros32)

    pre, s1, s2 = _tc_d1(acc[0, :N], acc[1, :N], ex[0, :N], ex[1, :N], x, W2,
                         b2.reshape(1, HID), Wr, b_conv.reshape(1, HID))
    return _tc_d2(pre, s1, s2, gamma.reshape(1, HID), beta.reshape(1, HID),
                  Wl1, bl1.reshape(1, HID // 2), Wl2, bl2.reshape(1, 1))


# speedup vs baseline: 1.0218x; 1.0218x over previous
import jax
import jax.numpy as jnp
from jax import lax
from jax.experimental import pallas as pl
from jax.experimental.pallas import tpu as pltpu
from jax.experimental.pallas import tpu_sc as plsc

N = 50000
E = 800000
HID = 64

NC, NS = 2, 16
PAY = 32
EXW = 8
SC_CHUNK = 512
N_CH_SCAT = 98
E_PAD = NS * N_CH_SCAT * SC_CHUNK
G_CHUNK = 512
N_CH_GATH = E_PAD // (NC * NS * G_CHUNK)
N_ACC = N + NS
ROWS_PT = N_ACC // NS

BLK_E = 8192
BLK_N = 2000

_SC_PARAMS = pltpu.CompilerParams(
    use_tc_tiling_on_sc=False, needs_layout_passes=False)


def _sc_gather_body(src_hbm, dst_hbm, xpad_hbm, zeros_hbm,
                    s_hbm, ex_hbm, acc, ibs, ibd, rbuf, sbuf, sem):
    cid = lax.axis_index("c")
    sid = lax.axis_index("s")
    wid = sid * NC + cid

    pltpu.sync_copy(zeros_hbm, acc.at[pl.ds(sid * ROWS_PT, ROWS_PT)])
    plsc.subcore_barrier()
    lanes = lax.iota(jnp.int32, 16)
    zl = jnp.zeros((16,), jnp.int32)

    def body(g, carry):
        ch = wid * N_CH_GATH + g
        r0 = ch * (G_CHUNK // 128)
        pltpu.sync_copy(src_hbm.at[pl.ds(r0, G_CHUNK // 128)], ibs)
        pltpu.sync_copy(dst_hbm.at[pl.ds(r0, G_CHUNK // 128)], ibd)
        for j in range(G_CHUNK // 128):
            pltpu.async_copy(xpad_hbm.at[ibs.at[j]],
                             rbuf.at[pl.ds(j * 128, 128)], sem).wait()
        for t in range(G_CHUNK // 16):
            sbuf[pl.ds(t * 16, 16)] = plsc.load_gather(
                rbuf, [t * 16 + lanes, zl])
        pltpu.sync_copy(sbuf, s_hbm.at[pl.ds(ch * G_CHUNK, G_CHUNK)])
        for j in range(G_CHUNK // 128):
            pltpu.sync_copy(rbuf.at[pl.ds(j * 128, 128)],
                            acc.at[ibd.at[j]], add=True)
        return carry

    lax.fori_loop(0, N_CH_GATH, body, 0)
    plsc.subcore_barrier()
    pltpu.sync_copy(acc.at[pl.ds(sid * ROWS_PT, ROWS_PT)],
                    ex_hbm.at[cid].at[pl.ds(sid * ROWS_PT, ROWS_PT)])


def _sc_scatter_body(p_hbm, dst_hbm, zeros_hbm, out_hbm, acc, pbuf, ibuf):
    cid = lax.axis_index("c")
    sid = lax.axis_index("s")

    pltpu.sync_copy(zeros_hbm, acc.at[pl.ds(sid * ROWS_PT, ROWS_PT)])
    plsc.subcore_barrier()

    def body(g, carry):
        e0 = (sid * N_CH_SCAT + g) * SC_CHUNK
        pltpu.sync_copy(p_hbm.at[cid].at[pl.ds(e0, SC_CHUNK)], pbuf)
        pltpu.sync_copy(dst_hbm.at[pl.ds(e0 // 128, SC_CHUNK // 128)], ibuf)
        for j in range(SC_CHUNK // 128):
            pltpu.sync_copy(pbuf.at[pl.ds(j * 128, 128)],
                            acc.at[ibuf.at[j]], add=True)
        return carry

    lax.fori_loop(0, N_CH_SCAT, body, 0)
    plsc.subcore_barrier()
    pltpu.sync_copy(acc.at[pl.ds(sid * ROWS_PT, ROWS_PT)],
                    out_hbm.at[cid].at[pl.ds(sid * ROWS_PT, ROWS_PT)])


_sc_cache = {}


def _sc_kernels():
    if "k" not in _sc_cache:
        mesh = plsc.VectorSubcoreMesh(
            core_axis_name="c", subcore_axis_name="s",
            num_cores=NC, num_subcores=NS)
        gather = pl.kernel(
            _sc_gather_body,
            out_type=[
                jax.ShapeDtypeStruct((E_PAD,), jnp.float32),
                jax.ShapeDtypeStruct((NC, N_ACC, EXW), jnp.float32),
            ],
            mesh=mesh,
            compiler_params=_SC_PARAMS,
            scratch_types=[
                pltpu.VMEM_SHARED((N_ACC, EXW), jnp.float32),
                pltpu.VMEM((G_CHUNK // 128, 128), jnp.int32),
                pltpu.VMEM((G_CHUNK // 128, 128), jnp.int32),
                pltpu.VMEM((G_CHUNK, EXW), jnp.float32),
                pltpu.VMEM((G_CHUNK,), jnp.float32),
                pltpu.SemaphoreType.DMA,
            ])
        scatter = pl.kernel(
            _sc_scatter_body,
            out_type=jax.ShapeDtypeStruct((NC, N_ACC, PAY), jnp.float32),
            mesh=mesh,
            compiler_params=_SC_PARAMS,
            scratch_types=[
                pltpu.VMEM_SHARED((N_ACC, PAY), jnp.float32),
                pltpu.VMEM((SC_CHUNK, PAY), jnp.float32),
                pltpu.VMEM((SC_CHUNK // 128, 128), jnp.int32),
            ])
        _sc_cache["k"] = (gather, scatter)
    return _sc_cache["k"]


Q = E_PAD // 4
BLK_R = 2048


def _edge_body(ea0, ea1, ea2, ea3, s4_ref, w1_ref, b1_ref, p_ref):
    lo, hi = [], []
    for k, ea_ref in enumerate((ea0, ea1, ea2, ea3)):
        z = jnp.dot(ea_ref[...], w1_ref[...],
                    preferred_element_type=jnp.float32)
        z = jnp.maximum(z + b1_ref[...], 0.0)
        sz = s4_ref[:, k:k + 1] * z
        lo.append(sz[:, :PAY])
        hi.append(sz[:, PAY:])
    p_ref[0] = jnp.concatenate(lo, axis=1)
    p_ref[1] = jnp.concatenate(hi, axis=1)


def _tc_edge(ea, s4t, W1, b1):
    nb = Q // BLK_R
    ea_specs = [pl.BlockSpec((BLK_R, 16), lambda i, k=k: (k * nb + i, 0))
                for k in range(4)]
    return pl.pallas_call(
        _edge_body,
        grid=(nb,),
        in_specs=ea_specs + [
            pl.BlockSpec((BLK_R, 4), lambda i: (i, 0)),
            pl.BlockSpec((16, HID), lambda i: (0, 0)),
            pl.BlockSpec((1, HID), lambda i: (0, 0)),
        ],
        out_specs=pl.BlockSpec((NC, BLK_R, 128), lambda i: (0, i, 0)),
        out_shape=jax.ShapeDtypeStruct((NC, Q, 128), jnp.float32),
    )(ea, ea, ea, ea, s4t, W1, b1)


def _d1_body(a0_ref, a1_ref, ex0_ref, ex1_ref, x_ref, w2_ref, b2_ref,
             wr_ref, bc_ref, pre_ref, sum_ref, sq_ref):
    A = jnp.concatenate([a0_ref[...], a1_ref[...]], axis=1)
    ex = ex0_ref[...] + ex1_ref[...]
    cs = ex[:, 0:1]
    deg = ex[:, 1:2]
    agg = jnp.dot(A, w2_ref[...], preferred_element_type=jnp.float32)
    agg = (agg + cs * b2_ref[...]) / jnp.maximum(deg, 1.0)
    pre = agg + x_ref[...] * wr_ref[...] + bc_ref[...]
    pre_ref[...] = pre

    @pl.when(pl.program_id(0) == 0)
    def _():
        sum_ref[...] = jnp.zeros_like(sum_ref)
        sq_ref[...] = jnp.zeros_like(sq_ref)

    sum_ref[...] += jnp.sum(pre, axis=0, keepdims=True)
    sq_ref[...] += jnp.sum(pre * pre, axis=0, keepdims=True)


def _tc_d1(A0, A1, EX0, EX1, x, W2, b2, Wr, bc):
    return pl.pallas_call(
        _d1_body,
        grid=(N // BLK_N,),
        in_specs=[
            pl.BlockSpec((BLK_N, PAY), lambda i: (i, 0)),
            pl.BlockSpec((BLK_N, PAY), lambda i: (i, 0)),
            pl.BlockSpec((BLK_N, EXW), lambda i: (i, 0)),
            pl.BlockSpec((BLK_N, EXW), lambda i: (i, 0)),
            pl.BlockSpec((BLK_N, 1), lambda i: (i, 0)),
            pl.BlockSpec((HID, HID), lambda i: (0, 0)),
            pl.BlockSpec((1, HID), lambda i: (0, 0)),
            pl.BlockSpec((1, HID), lambda i: (0, 0)),
            pl.BlockSpec((1, HID), lambda i: (0, 0)),
        ],
        out_specs=[
            pl.BlockSpec((BLK_N, HID), lambda i: (i, 0)),
            pl.BlockSpec((1, HID), lambda i: (0, 0)),
            pl.BlockSpec((1, HID), lambda i: (0, 0)),
        ],
        out_shape=[
            jax.ShapeDtypeStruct((N, HID), jnp.float32),
            jax.ShapeDtypeStruct((1, HID), jnp.float32),
            jax.ShapeDtypeStruct((1, HID), jnp.float32),
        ],
    )(A0, A1, EX0, EX1, x, W2, b2, Wr, bc)


def _d2_body(pre_ref, sum_ref, sq_ref, g_ref, be_ref, wl1_ref, bl1_ref,
             wl2_ref, bl2_ref, y_ref):
    mean = sum_ref[...] * (1.0 / N)
    var = sq_ref[...] * (1.0 / N) - mean * mean
    inv = lax.rsqrt(var + 1e-5)
    o = (pre_ref[...] - mean) * (inv * g_ref[...]) + be_ref[...]
    o = jnp.maximum(o, 0.0)
    l1 = jnp.dot(o, wl1_ref[...], preferred_element_type=jnp.float32)
    l1 = jax.nn.sigmoid(l1 + bl1_ref[...])
    y_ref[...] = (jnp.dot(l1, wl2_ref[...], preferred_element_type=jnp.float32)
                  + bl2_ref[...])


def _tc_d2(pre, s1, s2, gamma, beta, Wl1, bl1, Wl2, bl2):
    return pl.pallas_call(
        _d2_body,
        grid=(N // BLK_N,),
        in_specs=[
            pl.BlockSpec((BLK_N, HID), lambda i: (i, 0)),
            pl.BlockSpec((1, HID), lambda i: (0, 0)),
            pl.BlockSpec((1, HID), lambda i: (0, 0)),
            pl.BlockSpec((1, HID), lambda i: (0, 0)),
            pl.BlockSpec((1, HID), lambda i: (0, 0)),
            pl.BlockSpec((HID, HID // 2), lambda i: (0, 0)),
            pl.BlockSpec((1, HID // 2), lambda i: (0, 0)),
            pl.BlockSpec((HID // 2, 1), lambda i: (0, 0)),
            pl.BlockSpec((1, 1), lambda i: (0, 0)),
        ],
        out_specs=pl.BlockSpec((BLK_N, 1), lambda i: (i, 0)),
        out_shape=jax.ShapeDtypeStruct((N, 1), jnp.float32),
    )(pre, s1, s2, gamma, beta, Wl1, bl1, Wl2, bl2)


def kernel(x, edge_index, edge_attr, W1, b1, W2, b2, Wr, b_conv, gamma, beta,
           Wl1, bl1, Wl2, bl2):
    pad = E_PAD - E
    src = edge_index[0].astype(jnp.int32)
    dst = edge_index[1].astype(jnp.int32)
    src_p = jnp.concatenate([src, jnp.zeros((pad,), jnp.int32)])
    src_p = src_p.reshape(E_PAD // 128, 128)
    dst_p = jnp.concatenate([dst, jnp.full((pad,), N, jnp.int32)])
    dst_p = dst_p.reshape(E_PAD // 128, 128)
    xpad = jnp.concatenate(
        [x, jnp.ones((N, 1), jnp.float32), jnp.zeros((N, EXW - 2), jnp.float32)],
        axis=1)
    ea_p = jnp.concatenate(
        [edge_attr, jnp.zeros((pad, 16), jnp.float32)], axis=0)
    zeros8 = jnp.zeros((ROWS_PT, EXW), jnp.float32)
    zeros32 = jnp.zeros((ROWS_PT, PAY), jnp.float32)

    dst_sig = jnp.concatenate([dst, jnp.full((pad,), N, jnp.int32)])
    dst_sig = dst_sig.reshape(4, Q).T.reshape(E_PAD // 128, 128)

    sc_gather, sc_scatter = _sc_kernels()
    s_flat, ex = sc_gather(src_p, dst_p, xpad, zeros8)
    P = _tc_edge(ea_p, s_flat.reshape(4, Q).T, W1, b1.reshape(1, HID))
    acc = sc_scatter(P.reshape(NC, E_PAD, PAY), dst_sig, zeros32)

    pre, s1, s2 = _tc_d1(acc[0, :N], acc[1, :N], ex[0, :N], ex[1, :N], x, W2,
                         b2.reshape(1, HID), Wr, b_conv.reshape(1, HID))
    return _tc_d2(pre, s1, s2, gamma.reshape(1, HID), beta.reshape(1, HID),
                  Wl1, bl1.reshape(1, HID // 2), Wl2, bl2.reshape(1, 1))


# speedup vs baseline: 1.0600x; 1.0373x over previous
import jax
import jax.numpy as jnp
from jax import lax
from jax.experimental import pallas as pl
from jax.experimental.pallas import tpu as pltpu
from jax.experimental.pallas import tpu_sc as plsc

N = 50000
E = 800000
HID = 64

NC, NS = 2, 16
PAY = 32
EXW = 8
SC_CHUNK = 512
N_CH_SCAT = 98
E_PAD = NS * N_CH_SCAT * SC_CHUNK
G_CHUNK = 1792
N_CH_GATH = E_PAD // (NC * NS * G_CHUNK)
N_ACC = N + NS
ROWS_PT = N_ACC // NS

BLK_E = 8192
BLK_N = 2000

_SC_PARAMS = pltpu.CompilerParams(
    use_tc_tiling_on_sc=False, needs_layout_passes=False)


def _sc_gather_body(src_hbm, dst_hbm, xpad_hbm, zeros_hbm,
                    s_hbm, ex_hbm, acc, ibs, ibd, rbuf, sbuf, sem, sem2):
    cid = lax.axis_index("c")
    sid = lax.axis_index("s")
    wid = sid * NC + cid

    pltpu.sync_copy(zeros_hbm, acc.at[pl.ds(sid * ROWS_PT, ROWS_PT)])
    plsc.subcore_barrier()
    lanes = lax.iota(jnp.int32, 16)
    zl = jnp.zeros((16,), jnp.int32)

    def body(g, carry):
        ch = wid * N_CH_GATH + g
        r0 = ch * (G_CHUNK // 128)
        pltpu.sync_copy(src_hbm.at[pl.ds(r0, G_CHUNK // 128)], ibs)
        pltpu.sync_copy(dst_hbm.at[pl.ds(r0, G_CHUNK // 128)], ibd)
        gets = [pltpu.async_copy(xpad_hbm.at[ibs.at[j]],
                                 rbuf.at[pl.ds(j * 128, 128)], sem)
                for j in range(G_CHUNK // 128)]
        for d in gets:
            d.wait()
        for t in range(G_CHUNK // 16):
            sbuf[pl.ds(t * 16, 16)] = plsc.load_gather(
                rbuf, [t * 16 + lanes, zl])
        puts = [pltpu.async_copy(rbuf.at[pl.ds(j * 128, 128)],
                                 acc.at[ibd.at[j]], sem2, add=True)
                for j in range(G_CHUNK // 128)]
        pltpu.sync_copy(sbuf, s_hbm.at[pl.ds(ch * G_CHUNK, G_CHUNK)])
        for d in puts:
            d.wait()
        return carry

    lax.fori_loop(0, N_CH_GATH, body, 0)
    plsc.subcore_barrier()
    pltpu.sync_copy(acc.at[pl.ds(sid * ROWS_PT, ROWS_PT)],
                    ex_hbm.at[cid].at[pl.ds(sid * ROWS_PT, ROWS_PT)])


def _sc_scatter_body(p_hbm, dst_hbm, zeros_hbm, out_hbm, acc, pbuf, ibuf,
                     sem):
    cid = lax.axis_index("c")
    sid = lax.axis_index("s")

    pltpu.sync_copy(zeros_hbm, acc.at[pl.ds(sid * ROWS_PT, ROWS_PT)])
    plsc.subcore_barrier()

    def body(g, carry):
        e0 = (sid * N_CH_SCAT + g) * SC_CHUNK
        pltpu.sync_copy(p_hbm.at[cid].at[pl.ds(e0, SC_CHUNK)], pbuf)
        pltpu.sync_copy(dst_hbm.at[pl.ds(e0 // 128, SC_CHUNK // 128)], ibuf)
        puts = [pltpu.async_copy(pbuf.at[pl.ds(j * 128, 128)],
                                 acc.at[ibuf.at[j]], sem, add=True)
                for j in range(SC_CHUNK // 128)]
        for d in puts:
            d.wait()
        return carry

    lax.fori_loop(0, N_CH_SCAT, body, 0)
    plsc.subcore_barrier()
    pltpu.sync_copy(acc.at[pl.ds(sid * ROWS_PT, ROWS_PT)],
                    out_hbm.at[cid].at[pl.ds(sid * ROWS_PT, ROWS_PT)])


_sc_cache = {}


def _sc_kernels():
    if "k" not in _sc_cache:
        mesh = plsc.VectorSubcoreMesh(
            core_axis_name="c", subcore_axis_name="s",
            num_cores=NC, num_subcores=NS)
        gather = pl.kernel(
            _sc_gather_body,
            out_type=[
                jax.ShapeDtypeStruct((E_PAD,), jnp.float32),
                jax.ShapeDtypeStruct((NC, N_ACC, EXW), jnp.float32),
            ],
            mesh=mesh,
            compiler_params=_SC_PARAMS,
            scratch_types=[
                pltpu.VMEM_SHARED((N_ACC, EXW), jnp.float32),
                pltpu.VMEM((G_CHUNK // 128, 128), jnp.int32),
                pltpu.VMEM((G_CHUNK // 128, 128), jnp.int32),
                pltpu.VMEM((G_CHUNK, EXW), jnp.float32),
                pltpu.VMEM((G_CHUNK,), jnp.float32),
                pltpu.SemaphoreType.DMA,
                pltpu.SemaphoreType.DMA,
            ])
        scatter = pl.kernel(
            _sc_scatter_body,
            out_type=jax.ShapeDtypeStruct((NC, N_ACC, PAY), jnp.float32),
            mesh=mesh,
            compiler_params=_SC_PARAMS,
            scratch_types=[
                pltpu.VMEM_SHARED((N_ACC, PAY), jnp.float32),
                pltpu.VMEM((SC_CHUNK, PAY), jnp.float32),
                pltpu.VMEM((SC_CHUNK // 128, 128), jnp.int32),
                pltpu.SemaphoreType.DMA,
            ])
        _sc_cache["k"] = (gather, scatter)
    return _sc_cache["k"]


Q = E_PAD // 4
BLK_R = 2048


def _edge_body(ea0, ea1, ea2, ea3, s4_ref, w1_ref, b1_ref, p_ref):
    lo, hi = [], []
    for k, ea_ref in enumerate((ea0, ea1, ea2, ea3)):
        z = jnp.dot(ea_ref[...], w1_ref[...],
                    preferred_element_type=jnp.float32)
        z = jnp.maximum(z + b1_ref[...], 0.0)
        sz = s4_ref[:, k:k + 1] * z
        lo.append(sz[:, :PAY])
        hi.append(sz[:, PAY:])
    p_ref[0] = jnp.concatenate(lo, axis=1)
    p_ref[1] = jnp.concatenate(hi, axis=1)


def _tc_edge(ea, s4t, W1, b1):
    nb = Q // BLK_R
    ea_specs = [pl.BlockSpec((BLK_R, 16), lambda i, k=k: (k * nb + i, 0))
                for k in range(4)]
    return pl.pallas_call(
        _edge_body,
        grid=(nb,),
        in_specs=ea_specs + [
            pl.BlockSpec((BLK_R, 4), lambda i: (i, 0)),
            pl.BlockSpec((16, HID), lambda i: (0, 0)),
            pl.BlockSpec((1, HID), lambda i: (0, 0)),
        ],
        out_specs=pl.BlockSpec((NC, BLK_R, 128), lambda i: (0, i, 0)),
        out_shape=jax.ShapeDtypeStruct((NC, Q, 128), jnp.float32),
    )(ea, ea, ea, ea, s4t, W1, b1)


def _d1_body(a0_ref, a1_ref, ex0_ref, ex1_ref, x_ref, w2_ref, b2_ref,
             wr_ref, bc_ref, pre_ref, sum_ref, sq_ref):
    A = jnp.concatenate([a0_ref[...], a1_ref[...]], axis=1)
    ex = ex0_ref[...] + ex1_ref[...]
    cs = ex[:, 0:1]
    deg = ex[:, 1:2]
    agg = jnp.dot(A, w2_ref[...], preferred_element_type=jnp.float32)
    agg = (agg + cs * b2_ref[...]) / jnp.maximum(deg, 1.0)
    pre = agg + x_ref[...] * wr_ref[...] + bc_ref[...]
    pre_ref[...] = pre

    @pl.when(pl.program_id(0) == 0)
    def _():
        sum_ref[...] = jnp.zeros_like(sum_ref)
        sq_ref[...] = jnp.zeros_like(sq_ref)

    sum_ref[...] += jnp.sum(pre, axis=0, keepdims=True)
    sq_ref[...] += jnp.sum(pre * pre, axis=0, keepdims=True)


def _tc_d1(A0, A1, EX0, EX1, x, W2, b2, Wr, bc):
    return pl.pallas_call(
        _d1_body,
        grid=(N // BLK_N,),
        in_specs=[
            pl.BlockSpec((BLK_N, PAY), lambda i: (i, 0)),
            pl.BlockSpec((BLK_N, PAY), lambda i: (i, 0)),
            pl.BlockSpec((BLK_N, EXW), lambda i: (i, 0)),
            pl.BlockSpec((BLK_N, EXW), lambda i: (i, 0)),
            pl.BlockSpec((BLK_N, 1), lambda i: (i, 0)),
            pl.BlockSpec((HID, HID), lambda i: (0, 0)),
            pl.BlockSpec((1, HID), lambda i: (0, 0)),
            pl.BlockSpec((1, HID), lambda i: (0, 0)),
            pl.BlockSpec((1, HID), lambda i: (0, 0)),
        ],
        out_specs=[
            pl.BlockSpec((BLK_N, HID), lambda i: (i, 0)),
            pl.BlockSpec((1, HID), lambda i: (0, 0)),
            pl.BlockSpec((1, HID), lambda i: (0, 0)),
        ],
        out_shape=[
            jax.ShapeDtypeStruct((N, HID), jnp.float32),
            jax.ShapeDtypeStruct((1, HID), jnp.float32),
            jax.ShapeDtypeStruct((1, HID), jnp.float32),
        ],
    )(A0, A1, EX0, EX1, x, W2, b2, Wr, bc)


def _d2_body(pre_ref, sum_ref, sq_ref, g_ref, be_ref, wl1_ref, bl1_ref,
             wl2_ref, bl2_ref, y_ref):
    mean = sum_ref[...] * (1.0 / N)
    var = sq_ref[...] * (1.0 / N) - mean * mean
    inv = lax.rsqrt(var + 1e-5)
    o = (pre_ref[...] - mean) * (inv * g_ref[...]) + be_ref[...]
    o = jnp.maximum(o, 0.0)
    l1 = jnp.dot(o, wl1_ref[...], preferred_element_type=jnp.float32)
    l1 = jax.nn.sigmoid(l1 + bl1_ref[...])
    y_ref[...] = (jnp.dot(l1, wl2_ref[...], preferred_element_type=jnp.float32)
                  + bl2_ref[...])


def _tc_d2(pre, s1, s2, gamma, beta, Wl1, bl1, Wl2, bl2):
    return pl.pallas_call(
        _d2_body,
        grid=(N // BLK_N,),
        in_specs=[
            pl.BlockSpec((BLK_N, HID), lambda i: (i, 0)),
            pl.BlockSpec((1, HID), lambda i: (0, 0)),
            pl.BlockSpec((1, HID), lambda i: (0, 0)),
            pl.BlockSpec((1, HID), lambda i: (0, 0)),
            pl.BlockSpec((1, HID), lambda i: (0, 0)),
            pl.BlockSpec((HID, HID // 2), lambda i: (0, 0)),
            pl.BlockSpec((1, HID // 2), lambda i: (0, 0)),
            pl.BlockSpec((HID // 2, 1), lambda i: (0, 0)),
            pl.BlockSpec((1, 1), lambda i: (0, 0)),
        ],
        out_specs=pl.BlockSpec((BLK_N, 1), lambda i: (i, 0)),
        out_shape=jax.ShapeDtypeStruct((N, 1), jnp.float32),
    )(pre, s1, s2, gamma, beta, Wl1, bl1, Wl2, bl2)


def kernel(x, edge_index, edge_attr, W1, b1, W2, b2, Wr, b_conv, gamma, beta,
           Wl1, bl1, Wl2, bl2):
    pad = E_PAD - E
    src = edge_index[0].astype(jnp.int32)
    dst = edge_index[1].astype(jnp.int32)
    src_p = jnp.concatenate([src, jnp.zeros((pad,), jnp.int32)])
    src_p = src_p.reshape(E_PAD // 128, 128)
    dst_p = jnp.concatenate([dst, jnp.full((pad,), N, jnp.int32)])
    dst_p = dst_p.reshape(E_PAD // 128, 128)
    xpad = jnp.concatenate(
        [x, jnp.ones((N, 1), jnp.float32), jnp.zeros((N, EXW - 2), jnp.float32)],
        axis=1)
    ea_p = jnp.concatenate(
        [edge_attr, jnp.zeros((pad, 16), jnp.float32)], axis=0)
    zeros8 = jnp.zeros((ROWS_PT, EXW), jnp.float32)
    zeros32 = jnp.zeros((ROWS_PT, PAY), jnp.float32)

    dst_sig = jnp.concatenate([dst, jnp.full((pad,), N, jnp.int32)])
    dst_sig = dst_sig.reshape(4, Q).T.reshape(E_PAD // 128, 128)

    sc_gather, sc_scatter = _sc_kernels()
    s_flat, ex = sc_gather(src_p, dst_p, xpad, zeros8)
    P = _tc_edge(ea_p, s_flat.reshape(4, Q).T, W1, b1.reshape(1, HID))
    acc = sc_scatter(P.reshape(NC, E_PAD, PAY), dst_sig, zeros32)

    pre, s1, s2 = _tc_d1(acc[0, :N], acc[1, :N], ex[0, :N], ex[1, :N], x, W2,
                         b2.reshape(1, HID), Wr, b_conv.reshape(1, HID))
    return _tc_d2(pre, s1, s2, gamma.reshape(1, HID), beta.reshape(1, HID),
                  Wl1, bl1.reshape(1, HID // 2), Wl2, bl2.reshape(1, 1))
